# baseline (speedup 1.0000x reference)
import functools
import math

import jax
import jax.numpy as jnp
from jax import lax
from jax.experimental import pallas as pl
from jax.experimental.pallas import tpu as pltpu
from jax.experimental.pallas import tpu_sc as plsc

PI = math.pi
N_NODES = 10000
N_GRAPHS = 100
N_FIXED = 100
N_EDGES = 160000
NEG = -3.0e38

NW = 32
NLOC = 320
NLOCD = NLOC + 16
NPAD = NW * NLOC
CAPL = 1024
CAP = 16 * CAPL


def _leaky(v):
    return jnp.where(v > 0, v, 0.1 * v)


def _cycle(v):
    return v - 2.0 * PI * jnp.round(v / (2.0 * PI))


def _ab_body(h_ref, wd_ref, w2_ref, b_ref, a_ref, bb_ref):
    h = h_ref[...]
    a_ref[...] = jnp.dot(h, wd_ref[...], preferred_element_type=jnp.float32) + b_ref[...]
    bb_ref[...] = jnp.dot(h, w2_ref[...], preferred_element_type=jnp.float32)


def _tc_ab(h, wd, w2, b):
    n, f = h.shape
    fo = wd.shape[1]
    rows = 2000
    grid = (n // rows,)
    return pl.pallas_call(
        _ab_body,
        grid=grid,
        in_specs=[
            pl.BlockSpec((rows, f), lambda i: (i, 0)),
            pl.BlockSpec((f, fo), lambda i: (0, 0)),
            pl.BlockSpec((f, fo), lambda i: (0, 0)),
            pl.BlockSpec((1, fo), lambda i: (0, 0)),
        ],
        out_specs=[
            pl.BlockSpec((rows, fo), lambda i: (i, 0)),
            pl.BlockSpec((rows, fo), lambda i: (i, 0)),
        ],
        out_shape=[
            jax.ShapeDtypeStruct((n, fo), jnp.float32),
            jax.ShapeDtypeStruct((n, fo), jnp.float32),
        ],
    )(h, wd, w2, b.reshape(1, fo))


def _pool_body(h_ref, g_ref):
    h = h_ref[...]
    mean = jnp.sum(h, axis=1) / float(N_FIXED)
    mx = jnp.max(h, axis=1)
    g_ref[...] = jnp.concatenate([mean, mx], axis=1)


def _tc_pool(h):
    f = h.shape[1]
    h3 = h.reshape(N_GRAPHS, N_FIXED, f)
    return pl.pallas_call(
        _pool_body,
        in_specs=[pl.BlockSpec((N_GRAPHS, N_FIXED, f), lambda: (0, 0, 0))],
        out_specs=pl.BlockSpec((N_GRAPHS, 2 * f), lambda: (0, 0)),
        out_shape=jax.ShapeDtypeStruct((N_GRAPHS, 2 * f), jnp.float32),
    )(h3)


def _dense_body(g_ref, xmet_ref, fc1w, fc1b, metw, metb, fc2w, fc2b,
                d1w, d1b, dmw, dmb, d2w, d2b,
                z_ref, metbar_ref, xx_ref):
    xm = _leaky(jnp.dot(xmet_ref[...], metw[...],
                        preferred_element_type=jnp.float32) + metb[...])
    g1 = _leaky(jnp.dot(g_ref[...], fc1w[...],
                        preferred_element_type=jnp.float32) + fc1b[...])
    zin = jnp.concatenate([xm, g1], axis=1)
    z = jnp.dot(zin, fc2w[...], preferred_element_type=jnp.float32) + fc2b[...]
    z_ref[...] = z
    enc = _leaky(jnp.dot(z, d1w[...], preferred_element_type=jnp.float32) + d1b[...])
    xm2 = jnp.dot(enc, dmw[...], preferred_element_type=jnp.float32) + dmb[...]
    met_e = jnp.maximum(xm2[:, 0:1], 0.0)
    met_phi = _cycle(xm2[:, 1:2])
    metbar_ref[...] = jnp.concatenate([met_e, met_phi], axis=1)
    xx_ref[...] = _leaky(jnp.dot(enc, d2w[...],
                                 preferred_element_type=jnp.float32) + d2b[...])


def _tc_dense(g, x_met, p):
    fc1w, fc1b = p['enc_fc1']
    metw, metb = p['enc_met_fc1']
    fc2w, fc2b = p['enc_fc2']
    d1w, d1b = p['dec_fc1']
    dmw, dmb = p['dec_met_fc1']
    d2w, d2b = p['dec_fc2']
    dmw_p = jnp.zeros((264, 2), jnp.float32).at[:8].set(dmw)
    d2w_p = jnp.zeros((264, d2w.shape[1]), jnp.float32).at[8:].set(d2w)
    outs = pl.pallas_call(
        _dense_body,
        in_specs=[pl.BlockSpec(x.shape, lambda: tuple([0] * x.ndim)) for x in (
            g, x_met, fc1w, fc1b.reshape(1, -1), metw, metb.reshape(1, -1),
            fc2w, fc2b.reshape(1, -1), d1w, d1b.reshape(1, -1),
            dmw_p, dmb.reshape(1, -1), d2w_p, d2b.reshape(1, -1))],
        out_specs=[
            pl.BlockSpec((N_GRAPHS, 128), lambda: (0, 0)),
            pl.BlockSpec((N_GRAPHS, 2), lambda: (0, 0)),
            pl.BlockSpec((N_GRAPHS, d2w.shape[1]), lambda: (0, 0)),
        ],
        out_shape=[
            jax.ShapeDtypeStruct((N_GRAPHS, 128), jnp.float32),
            jax.ShapeDtypeStruct((N_GRAPHS, 2), jnp.float32),
            jax.ShapeDtypeStruct((N_GRAPHS, d2w.shape[1]), jnp.float32),
        ],
    )(g, x_met, fc1w, fc1b.reshape(1, -1), metw, metb.reshape(1, -1),
      fc2w, fc2b.reshape(1, -1), d1w, d1b.reshape(1, -1),
      dmw_p, dmb.reshape(1, -1), d2w_p, d2b.reshape(1, -1))
    return outs


def _heads_body(o_ref, out_ref):
    o = o_ref[...]
    cat = o[:, 0:4]
    m = jnp.max(cat, axis=1, keepdims=True)
    lse = m + jnp.log(jnp.sum(jnp.exp(cat - m), axis=1, keepdims=True))
    x_cat = cat - lse
    x_ept = jnp.maximum(o[:, 4:6], 0.0)
    x_eta = 5.0 * jnp.tanh(o[:, 6:7])
    x_phi = PI * jnp.tanh(_cycle(o[:, 7:8]))
    out_ref[...] = jnp.concatenate([x_cat, x_ept, x_eta, x_phi], axis=1)


def _tc_heads(o16):
    rows = 2000
    return pl.pallas_call(
        _heads_body,
        grid=(N_NODES // rows,),
        in_specs=[pl.BlockSpec((rows, 128), lambda i: (i, 0))],
        out_specs=pl.BlockSpec((rows, 8), lambda i: (i, 0)),
        out_shape=jax.ShapeDtypeStruct((N_NODES, 8), jnp.float32),
    )(o16)


_MESH = dict(core_axis_name="c", subcore_axis_name="s")


def _wid():
    return lax.axis_index("s") * 2 + lax.axis_index("c")


def _sc_conv(f, act):
    chunks = f // 16
    unr = chunks
    n_c = 1

    @functools.partial(
        pl.kernel,
        out_type=jax.ShapeDtypeStruct((NPAD * f,), jnp.float32),
        mesh=plsc.VectorSubcoreMesh(**_MESH),
        scratch_types=[
            pltpu.VMEM((NLOCD * f,), jnp.float32),
            pltpu.VMEM((16, f), jnp.float32),
            pltpu.VMEM((16, f), jnp.float32),
            pltpu.VMEM((CAPL,), jnp.int32),
            pltpu.VMEM((CAPL,), jnp.int32),
            pltpu.VMEM((NLOC,), jnp.int32),
            pltpu.VMEM((16,), jnp.int32),
            pltpu.SemaphoreType.DMA,
        ],
    )
    def k(a_h, b_h, gidx_h, srcl_h, dstl_h, startv_h, endv_h, out_h,
          mv, rows, arows, sidx, didx, gidxv, cv, sem):
        wid = _wid()
        lo = wid * NLOC
        neg = jnp.full((16,), NEG, jnp.float32)
        ii = lax.iota(jnp.int32, 16)

        def initb(i, _):
            mv[pl.ds(i * 16, 16)] = neg
            return 0

        lax.fori_loop(0, NLOCD * f // 16, initb, 0)
        pltpu.sync_copy(startv_h.at[pl.ds(wid * 16, 16)], cv)
        start = cv[...][0]
        pltpu.sync_copy(endv_h.at[pl.ds(wid * 16, 16)], cv)
        end = cv[...][0]
        pltpu.sync_copy(gidx_h.at[pl.ds(lo, NLOC)], gidxv)

        blk = 1024
        abase = (start // 16) * 16
        nb_all = (end - abase + 15) // 16

        def outer(ob, _):
            pltpu.sync_copy(srcl_h.at[pl.ds(abase + ob * blk, blk)], sidx)
            pltpu.sync_copy(dstl_h.at[pl.ds(abase + ob * blk, blk)], didx)
            nb = jnp.minimum(nb_all - ob * (blk // 16), blk // 16)

            def batch(g, _):
                pltpu.async_copy(
                    b_h.at[sidx.at[pl.ds(g * 16, 16)]], rows, sem).wait()
                dv = didx[pl.ds(g * 16, 16)]
                kvec = abase + ob * blk + g * 16 + ii
                valid = (kvec >= start) & (kvec < end)
                drs = jnp.where(valid, dv - lo, NLOC)
                for t in range(16):
                    off = drs[t] * f

                    def rmw(c, _):
                        o = off + c * (16 * unr)
                        for u in range(unr):
                            cur = mv[pl.ds(o + u * 16, 16)]
                            mv[pl.ds(o + u * 16, 16)] = jnp.maximum(
                                cur, rows[t, pl.ds(c * (16 * unr) + u * 16, 16)])
                        return 0

                    lax.fori_loop(0, n_c, rmw, 0)
                return 0

            lax.fori_loop(0, nb, batch, 0)
            return 0

        lax.fori_loop(0, (nb_all + (blk // 16) - 1) // (blk // 16), outer, 0)

        def outb(db, _):
            pltpu.async_copy(
                a_h.at[gidxv.at[pl.ds(db * 16, 16)]], arows, sem).wait()
            for e in range(16):
                off = (db * 16 + e) * f

                def comb(c, _):
                    o = off + c * (16 * unr)
                    for u in range(unr):
                        mrow = mv[pl.ds(o + u * 16, 16)]
                        arow = arows[e, pl.ds(c * (16 * unr) + u * 16, 16)]
                        fin = mrow > -1.0e37
                        v = arow + jnp.where(fin, mrow, 0.0)
                        if act:
                            v = jnp.where(v > 0, v, 0.1 * v)
                        mv[pl.ds(o + u * 16, 16)] = jnp.where(fin, v, 0.0)
                    return 0

                lax.fori_loop(0, n_c, comb, 0)
            return 0

        lax.fori_loop(0, NLOC // 16, outb, 0)
        pltpu.sync_copy(mv.at[pl.ds(0, NLOC * f)],
                        out_h.at[pl.ds(lo * f, NLOC * f)])

    return k


def _split(p):
    w, b = p
    f = w.shape[0] // 2
    return w[:f] - w[f:], w[f:], b


def kernel(x, x_met, edge_index, batch, params):
    src = edge_index[0].astype(jnp.int32)
    dst = edge_index[1].astype(jnp.int32)

    idx = jax.random.randint(jax.random.key(42), (N_GRAPHS, N_FIXED), 0, N_FIXED)
    p = (jnp.arange(N_GRAPHS, dtype=jnp.int32)[:, None] * N_FIXED
         + idx.astype(jnp.int32)).reshape(-1)
    p_pad = jnp.zeros((NPAD,), jnp.int32).at[:N_NODES].set(p)
    ident = jnp.minimum(jnp.arange(NPAD, dtype=jnp.int32), N_NODES - 1)

    E_PAD = N_EDGES + 3840
    order = jnp.argsort(dst)
    dsts = dst[order]
    srcs = src[order]
    srcps = p[srcs]
    bounds = jnp.arange(NW + 1, dtype=jnp.int32) * NLOC
    cuts = jnp.searchsorted(dsts, bounds).astype(jnp.int32)
    startv = jnp.repeat(cuts[:-1], 16)
    endv = jnp.repeat(cuts[1:], 16)
    zpad = jnp.zeros((E_PAD - N_EDGES,), jnp.int32)
    dsts = jnp.concatenate([dsts, zpad])
    srcs = jnp.concatenate([srcs, zpad])
    srcps = jnp.concatenate([srcps, zpad])

    def conv(h, name, act, srclist, gidx, fo_pad=None):
        wd, w2, b = _split(params[name])
        if fo_pad is not None:
            fo = wd.shape[1]
            wd = jnp.zeros((wd.shape[0], fo_pad), jnp.float32).at[:, :fo].set(wd)
            w2 = jnp.zeros((w2.shape[0], fo_pad), jnp.float32).at[:, :fo].set(w2)
            b = jnp.zeros((fo_pad,), jnp.float32).at[:fo].set(b)
        a, bb = _tc_ab(h, wd, w2, b)
        f = wd.shape[1]
        out = _sc_conv(f, act)(a, bb, gidx, srclist, dsts, startv, endv)
        return out.reshape(NPAD, f)[:N_NODES]

    h1 = conv(x.astype(jnp.float32), 'enc_conv0', True, srcs, ident)
    h2 = conv(h1, 'enc_conv1', True, srcs, ident)

    g = _tc_pool(h2)
    z, x_met_bar, xx = _tc_dense(g, x_met, params)
    xx = xx.reshape(N_NODES, 256)

    h4 = conv(xx, 'dec_conv0', True, srcps, p_pad)
    h5 = conv(h4, 'dec_conv1', True, srcs, ident)
    o16 = conv(h5, 'dec_conv2', False, srcs, ident, fo_pad=128)

    x_bar = _tc_heads(o16)
    return (x_bar, x_met_bar, z)


# speedup vs baseline: 1.2138x; 1.2138x over previous
import functools
import math

import jax
import jax.numpy as jnp
from jax import lax
from jax.experimental import pallas as pl
from jax.experimental.pallas import tpu as pltpu
from jax.experimental.pallas import tpu_sc as plsc

PI = math.pi
N_NODES = 10000
N_GRAPHS = 100
N_FIXED = 100
N_EDGES = 160000
NEG = -3.0e38

NW = 32
NLOC = 320
NLOCD = NLOC + 16
NPAD = NW * NLOC
CAPL = 1024
CAP = 16 * CAPL


def _leaky(v):
    return jnp.where(v > 0, v, 0.1 * v)


def _cycle(v):
    return v - 2.0 * PI * jnp.round(v / (2.0 * PI))


def _ab_body(h_ref, wd_ref, w2_ref, b_ref, a_ref, bb_ref):
    h = h_ref[...]
    a_ref[...] = jnp.dot(h, wd_ref[...], preferred_element_type=jnp.float32) + b_ref[...]
    bb_ref[...] = jnp.dot(h, w2_ref[...], preferred_element_type=jnp.float32)


def _tc_ab(h, wd, w2, b):
    n, f = h.shape
    fo = wd.shape[1]
    rows = 2000
    grid = (n // rows,)
    return pl.pallas_call(
        _ab_body,
        grid=grid,
        in_specs=[
            pl.BlockSpec((rows, f), lambda i: (i, 0)),
            pl.BlockSpec((f, fo), lambda i: (0, 0)),
            pl.BlockSpec((f, fo), lambda i: (0, 0)),
            pl.BlockSpec((1, fo), lambda i: (0, 0)),
        ],
        out_specs=[
            pl.BlockSpec((rows, fo), lambda i: (i, 0)),
            pl.BlockSpec((rows, fo), lambda i: (i, 0)),
        ],
        out_shape=[
            jax.ShapeDtypeStruct((n, fo), jnp.float32),
            jax.ShapeDtypeStruct((n, fo), jnp.float32),
        ],
    )(h, wd, w2, b.reshape(1, fo))


def _pool_body(h_ref, g_ref):
    h = h_ref[...]
    mean = jnp.sum(h, axis=1) / float(N_FIXED)
    mx = jnp.max(h, axis=1)
    g_ref[...] = jnp.concatenate([mean, mx], axis=1)


def _tc_pool(h):
    f = h.shape[1]
    h3 = h.reshape(N_GRAPHS, N_FIXED, f)
    return pl.pallas_call(
        _pool_body,
        in_specs=[pl.BlockSpec((N_GRAPHS, N_FIXED, f), lambda: (0, 0, 0))],
        out_specs=pl.BlockSpec((N_GRAPHS, 2 * f), lambda: (0, 0)),
        out_shape=jax.ShapeDtypeStruct((N_GRAPHS, 2 * f), jnp.float32),
    )(h3)


def _dense_body(g_ref, xmet_ref, fc1w, fc1b, metw, metb, fc2w, fc2b,
                d1w, d1b, dmw, dmb, d2w, d2b,
                z_ref, metbar_ref, xx_ref):
    xm = _leaky(jnp.dot(xmet_ref[...], metw[...],
                        preferred_element_type=jnp.float32) + metb[...])
    g1 = _leaky(jnp.dot(g_ref[...], fc1w[...],
                        preferred_element_type=jnp.float32) + fc1b[...])
    zin = jnp.concatenate([xm, g1], axis=1)
    z = jnp.dot(zin, fc2w[...], preferred_element_type=jnp.float32) + fc2b[...]
    z_ref[...] = z
    enc = _leaky(jnp.dot(z, d1w[...], preferred_element_type=jnp.float32) + d1b[...])
    xm2 = jnp.dot(enc, dmw[...], preferred_element_type=jnp.float32) + dmb[...]
    met_e = jnp.maximum(xm2[:, 0:1], 0.0)
    met_phi = _cycle(xm2[:, 1:2])
    metbar_ref[...] = jnp.concatenate([met_e, met_phi], axis=1)
    xx_ref[...] = _leaky(jnp.dot(enc, d2w[...],
                                 preferred_element_type=jnp.float32) + d2b[...])


def _tc_dense(g, x_met, p):
    fc1w, fc1b = p['enc_fc1']
    metw, metb = p['enc_met_fc1']
    fc2w, fc2b = p['enc_fc2']
    d1w, d1b = p['dec_fc1']
    dmw, dmb = p['dec_met_fc1']
    d2w, d2b = p['dec_fc2']
    dmw_p = jnp.zeros((264, 2), jnp.float32).at[:8].set(dmw)
    d2w_p = jnp.zeros((264, d2w.shape[1]), jnp.float32).at[8:].set(d2w)
    outs = pl.pallas_call(
        _dense_body,
        in_specs=[pl.BlockSpec(x.shape, lambda: tuple([0] * x.ndim)) for x in (
            g, x_met, fc1w, fc1b.reshape(1, -1), metw, metb.reshape(1, -1),
            fc2w, fc2b.reshape(1, -1), d1w, d1b.reshape(1, -1),
            dmw_p, dmb.reshape(1, -1), d2w_p, d2b.reshape(1, -1))],
        out_specs=[
            pl.BlockSpec((N_GRAPHS, 128), lambda: (0, 0)),
            pl.BlockSpec((N_GRAPHS, 2), lambda: (0, 0)),
            pl.BlockSpec((N_GRAPHS, d2w.shape[1]), lambda: (0, 0)),
        ],
        out_shape=[
            jax.ShapeDtypeStruct((N_GRAPHS, 128), jnp.float32),
            jax.ShapeDtypeStruct((N_GRAPHS, 2), jnp.float32),
            jax.ShapeDtypeStruct((N_GRAPHS, d2w.shape[1]), jnp.float32),
        ],
    )(g, x_met, fc1w, fc1b.reshape(1, -1), metw, metb.reshape(1, -1),
      fc2w, fc2b.reshape(1, -1), d1w, d1b.reshape(1, -1),
      dmw_p, dmb.reshape(1, -1), d2w_p, d2b.reshape(1, -1))
    return outs


def _heads_body(o_ref, out_ref):
    o = o_ref[...]
    cat = o[:, 0:4]
    m = jnp.max(cat, axis=1, keepdims=True)
    lse = m + jnp.log(jnp.sum(jnp.exp(cat - m), axis=1, keepdims=True))
    x_cat = cat - lse
    x_ept = jnp.maximum(o[:, 4:6], 0.0)
    x_eta = 5.0 * jnp.tanh(o[:, 6:7])
    x_phi = PI * jnp.tanh(_cycle(o[:, 7:8]))
    out_ref[...] = jnp.concatenate([x_cat, x_ept, x_eta, x_phi], axis=1)


def _tc_heads(o16):
    rows = 2000
    return pl.pallas_call(
        _heads_body,
        grid=(N_NODES // rows,),
        in_specs=[pl.BlockSpec((rows, 128), lambda i: (i, 0))],
        out_specs=pl.BlockSpec((rows, 8), lambda i: (i, 0)),
        out_shape=jax.ShapeDtypeStruct((N_NODES, 8), jnp.float32),
    )(o16)


_MESH = dict(core_axis_name="c", subcore_axis_name="s")


def _wid():
    return lax.axis_index("s") * 2 + lax.axis_index("c")


def _sc_conv(f, act):
    chunks = f // 16
    unr = chunks
    n_c = 1

    @functools.partial(
        pl.kernel,
        out_type=jax.ShapeDtypeStruct((NPAD * f,), jnp.float32),
        mesh=plsc.VectorSubcoreMesh(**_MESH),
        scratch_types=[
            pltpu.VMEM((NLOCD * f,), jnp.float32),
            pltpu.VMEM((16, f), jnp.float32),
            pltpu.VMEM((16, f), jnp.float32),
            pltpu.VMEM((16, f), jnp.float32),
            pltpu.VMEM((CAPL,), jnp.int32),
            pltpu.VMEM((CAPL,), jnp.int32),
            pltpu.VMEM((NLOC,), jnp.int32),
            pltpu.VMEM((16,), jnp.int32),
            pltpu.SemaphoreType.DMA,
            pltpu.SemaphoreType.DMA,
        ],
    )
    def k(a_h, b_h, gidx_h, srcl_h, dstl_h, startv_h, endv_h, out_h,
          mv, rows, rows1, arows, sidx, didx, gidxv, cv, sem, sem1):
        wid = _wid()
        lo = wid * NLOC
        neg = jnp.full((16,), NEG, jnp.float32)
        ii = lax.iota(jnp.int32, 16)

        def initb(i, _):
            mv[pl.ds(i * 16, 16)] = neg
            return 0

        lax.fori_loop(0, NLOCD * f // 16, initb, 0)
        pltpu.sync_copy(startv_h.at[pl.ds(wid * 16, 16)], cv)
        start = cv[...][0]
        pltpu.sync_copy(endv_h.at[pl.ds(wid * 16, 16)], cv)
        end = cv[...][0]
        pltpu.sync_copy(gidx_h.at[pl.ds(lo, NLOC)], gidxv)

        blk = 1024
        abase = (start // 16) * 16
        nb_all = (end - abase + 15) // 16

        def rmw_batch(g, ob, buf):
            dv = didx[pl.ds(g * 16, 16)]
            kvec = abase + ob * blk + g * 16 + ii
            valid = (kvec >= start) & (kvec < end)
            drs = jnp.where(valid, dv - lo, NLOC)
            for t in range(16):
                off = drs[t] * f
                for c in range(chunks):
                    cur = mv[pl.ds(off + c * 16, 16)]
                    mv[pl.ds(off + c * 16, 16)] = jnp.maximum(
                        cur, buf[t, pl.ds(c * 16, 16)])

        def fire(g, buf, s):
            pltpu.async_copy(b_h.at[sidx.at[pl.ds(g * 16, 16)]], buf, s)

        def drain(buf, s):
            pltpu.make_async_copy(b_h.at[sidx.at[pl.ds(0, 16)]], buf, s).wait()

        def outer(ob, _):
            pltpu.sync_copy(srcl_h.at[pl.ds(abase + ob * blk, blk)], sidx)
            pltpu.sync_copy(dstl_h.at[pl.ds(abase + ob * blk, blk)], didx)
            nb = jnp.minimum(nb_all - ob * (blk // 16), blk // 16)
            fire(0, rows, sem)

            def pair(pg, _):
                g0 = pg * 2
                g1 = g0 + 1
                drain(rows, sem)

                @pl.when(g1 < nb)
                def _():
                    fire(g1, rows1, sem1)

                rmw_batch(g0, ob, rows)

                @pl.when(g1 < nb)
                def _():
                    drain(rows1, sem1)

                    @pl.when(g1 + 1 < nb)
                    def _():
                        fire(g1 + 1, rows, sem)

                    rmw_batch(g1, ob, rows1)
                return 0

            lax.fori_loop(0, (nb + 1) // 2, pair, 0)
            return 0

        lax.fori_loop(0, (nb_all + (blk // 16) - 1) // (blk // 16), outer, 0)

        def outb(db, _):
            pltpu.async_copy(
                a_h.at[gidxv.at[pl.ds(db * 16, 16)]], arows, sem).wait()
            for e in range(16):
                off = (db * 16 + e) * f

                def comb(c, _):
                    o = off + c * (16 * unr)
                    for u in range(unr):
                        mrow = mv[pl.ds(o + u * 16, 16)]
                        arow = arows[e, pl.ds(c * (16 * unr) + u * 16, 16)]
                        fin = mrow > -1.0e37
                        v = arow + jnp.where(fin, mrow, 0.0)
                        if act:
                            v = jnp.where(v > 0, v, 0.1 * v)
                        mv[pl.ds(o + u * 16, 16)] = jnp.where(fin, v, 0.0)
                    return 0

                lax.fori_loop(0, n_c, comb, 0)
            return 0

        lax.fori_loop(0, NLOC // 16, outb, 0)
        pltpu.sync_copy(mv.at[pl.ds(0, NLOC * f)],
                        out_h.at[pl.ds(lo * f, NLOC * f)])

    return k


def _split(p):
    w, b = p
    f = w.shape[0] // 2
    return w[:f] - w[f:], w[f:], b


def kernel(x, x_met, edge_index, batch, params):
    src = edge_index[0].astype(jnp.int32)
    dst = edge_index[1].astype(jnp.int32)

    idx = jax.random.randint(jax.random.key(42), (N_GRAPHS, N_FIXED), 0, N_FIXED)
    p = (jnp.arange(N_GRAPHS, dtype=jnp.int32)[:, None] * N_FIXED
         + idx.astype(jnp.int32)).reshape(-1)
    p_pad = jnp.zeros((NPAD,), jnp.int32).at[:N_NODES].set(p)
    ident = jnp.minimum(jnp.arange(NPAD, dtype=jnp.int32), N_NODES - 1)

    E_PAD = N_EDGES + 3840
    order = jnp.argsort(dst)
    dsts = dst[order]
    srcs = src[order]
    srcps = p[srcs]
    bounds = jnp.arange(NW + 1, dtype=jnp.int32) * NLOC
    cuts = jnp.searchsorted(dsts, bounds).astype(jnp.int32)
    startv = jnp.repeat(cuts[:-1], 16)
    endv = jnp.repeat(cuts[1:], 16)
    zpad = jnp.zeros((E_PAD - N_EDGES,), jnp.int32)
    dsts = jnp.concatenate([dsts, zpad])
    srcs = jnp.concatenate([srcs, zpad])
    srcps = jnp.concatenate([srcps, zpad])

    def conv(h, name, act, srclist, gidx, fo_pad=None):
        wd, w2, b = _split(params[name])
        if fo_pad is not None:
            fo = wd.shape[1]
            wd = jnp.zeros((wd.shape[0], fo_pad), jnp.float32).at[:, :fo].set(wd)
            w2 = jnp.zeros((w2.shape[0], fo_pad), jnp.float32).at[:, :fo].set(w2)
            b = jnp.zeros((fo_pad,), jnp.float32).at[:fo].set(b)
        a, bb = _tc_ab(h, wd, w2, b)
        f = wd.shape[1]
        out = _sc_conv(f, act)(a, bb, gidx, srclist, dsts, startv, endv)
        return out.reshape(NPAD, f)[:N_NODES]

    h1 = conv(x.astype(jnp.float32), 'enc_conv0', True, srcs, ident)
    h2 = conv(h1, 'enc_conv1', True, srcs, ident)

    g = _tc_pool(h2)
    z, x_met_bar, xx = _tc_dense(g, x_met, params)
    xx = xx.reshape(N_NODES, 256)

    h4 = conv(xx, 'dec_conv0', True, srcps, p_pad)
    h5 = conv(h4, 'dec_conv1', True, srcs, ident)
    o16 = conv(h5, 'dec_conv2', False, srcs, ident, fo_pad=128)

    x_bar = _tc_heads(o16)
    return (x_bar, x_met_bar, z)
